# baseline (speedup 1.0000x reference)
import functools

import jax
import jax.numpy as jnp
from jax import lax
from jax.experimental import pallas as pl
from jax.experimental.pallas import tpu as pltpu
from jax.experimental.pallas import tpu_sc as plsc

N_REAL = 10000
N_PAD = 10240
N_EDGES = 320000
N_GRAPHS = 512
IN_DIM = 128
H = 64

NC, NS = 2, 16
NW = NC * NS
CHUNK = 128
NCHUNK = 80
NCH_TOT = 2560
SPLIT0 = 120
SPLIT1 = 40
GUARD = SPLIT0 - SPLIT1
E_PAD = (NCH_TOT + GUARD) * CHUNK
ROWS_PER_TILE = N_PAD // NS
PAD_IDX = 10016
NBUF = 5

_MESH = plsc.VectorSubcoreMesh(
    core_axis_name="c", subcore_axis_name="s", num_cores=NC, num_subcores=NS
)
_SC_PARAMS = pltpu.CompilerParams(use_tc_tiling_on_sc=False)


@functools.partial(
    pl.kernel,
    out_type=jax.ShapeDtypeStruct((NC, N_PAD), jnp.float32),
    mesh=_MESH,
    scratch_types=[
        pltpu.VMEM((NCHUNK, CHUNK), jnp.int32),
        pltpu.VMEM((CHUNK,), jnp.float32),
        pltpu.VMEM_SHARED((N_PAD,), jnp.float32),
    ],
    compiler_params=_SC_PARAMS,
)
def _sc_deg(edst_hbm, out_hbm, didx, buf, degsp):
    c = lax.axis_index("c")
    s = lax.axis_index("s")
    wid = c * NS + s
    pltpu.sync_copy(edst_hbm.at[pl.ds(wid * NCHUNK, NCHUNK)], didx)

    def _zero(i, carry):
        buf[pl.ds(i * 16, 16)] = jnp.zeros((16,), jnp.float32)
        return carry

    lax.fori_loop(0, CHUNK // 16, _zero, 0)
    r0 = s * ROWS_PER_TILE
    for j in range(ROWS_PER_TILE // CHUNK):
        pltpu.sync_copy(buf, degsp.at[pl.ds(r0 + j * CHUNK, CHUNK)])
    plsc.subcore_barrier()

    def _ones(i, carry):
        buf[pl.ds(i * 16, 16)] = jnp.ones((16,), jnp.float32)
        return carry

    lax.fori_loop(0, CHUNK // 16, _ones, 0)

    def _scatter(k, carry):
        pltpu.sync_copy(buf, degsp.at[didx.at[k]], add=True)
        return carry

    lax.fori_loop(0, NCHUNK, _scatter, 0)
    plsc.subcore_barrier()
    for j in range(ROWS_PER_TILE // CHUNK):
        pltpu.sync_copy(degsp.at[pl.ds(r0 + j * CHUNK, CHUNK)], buf)
        pltpu.sync_copy(buf, out_hbm.at[c, pl.ds(r0 + j * CHUNK, CHUNK)])


@functools.partial(
    pl.kernel,
    out_type=jax.ShapeDtypeStruct((NC, N_PAD, H), jnp.float32),
    mesh=_MESH,
    scratch_types=(
        [
            pltpu.VMEM((SPLIT0, CHUNK), jnp.int32),
            pltpu.VMEM((SPLIT0, CHUNK), jnp.int32),
        ]
        + [pltpu.VMEM((CHUNK, H), jnp.float32) for _ in range(NBUF)]
        + [
            pltpu.VMEM_SHARED((N_PAD, H), jnp.float32),
        ]
        + [pltpu.SemaphoreType.DMA for _ in range(NBUF)]
    ),
    compiler_params=_SC_PARAMS,
)
def _sc_agg(xs_hbm, esrc_hbm, edst_hbm, out_hbm, sidx, didx, *rest):
    rows = list(rest[:NBUF])
    aggsp = rest[NBUF]
    gsem = list(rest[NBUF + 1:])
    c = lax.axis_index("c")
    s = lax.axis_index("s")
    start = s * (SPLIT0 + SPLIT1) + jnp.where(c == 0, 0, SPLIT0)
    n_my = jnp.where(c == 0, SPLIT0, SPLIT1)
    pltpu.sync_copy(esrc_hbm.at[pl.ds(start, SPLIT0)], sidx)
    pltpu.sync_copy(edst_hbm.at[pl.ds(start, SPLIT0)], didx)

    def _zero(i, carry):
        for j in range(H // 16):
            rows[0][i, pl.ds(j * 16, 16)] = jnp.zeros((16,), jnp.float32)
        return carry

    lax.fori_loop(0, CHUNK, _zero, 0)
    r0 = s * ROWS_PER_TILE
    for j in range(ROWS_PER_TILE // CHUNK):
        pltpu.sync_copy(rows[0], aggsp.at[pl.ds(r0 + j * CHUNK, CHUNK)])
    plsc.subcore_barrier()

    def _group(k4, carry):
        for b in range(NBUF):
            k = k4 * NBUF + b
            kprev = k - NBUF

            @pl.when(kprev >= 0)
            def _():
                pltpu.make_async_copy(
                    xs_hbm.at[sidx.at[kprev]], rows[b], gsem[b]
                ).wait()
                pltpu.sync_copy(rows[b], aggsp.at[didx.at[kprev]], add=True)

            @pl.when(k < n_my)
            def _():
                pltpu.async_copy(xs_hbm.at[sidx.at[k]], rows[b], gsem[b])

        return carry

    lax.fori_loop(0, n_my // NBUF + 1, _group, 0)

    plsc.subcore_barrier()
    for j in range(ROWS_PER_TILE // CHUNK):
        pltpu.sync_copy(aggsp.at[pl.ds(r0 + j * CHUNK, CHUNK)], rows[0])
        pltpu.sync_copy(rows[0], out_hbm.at[c, pl.ds(r0 + j * CHUNK, CHUNK)])


def _tc_a_body(v_ref, w1_ref, degp_ref, xs_ref, dinv_ref):
    d = degp_ref[0, :, 0:1] + degp_ref[1, :, 0:1] + 1.0
    dinv = lax.rsqrt(d)
    dinv_ref[...] = dinv
    xw = jnp.dot(v_ref[...], w1_ref[...], preferred_element_type=jnp.float32)
    xs_ref[...] = xw * dinv


def _tc_a(vp, w1, degp):
    return pl.pallas_call(
        _tc_a_body,
        out_shape=(
            jax.ShapeDtypeStruct((N_PAD, H), jnp.float32),
            jax.ShapeDtypeStruct((N_PAD, 1), jnp.float32),
        ),
    )(vp, w1, degp)


def _tc_mid_body(aggp_ref, xs_ref, dinv_ref, b_ref, w_ref, post_ref, out_ref):
    agg = aggp_ref[0] + aggp_ref[1]
    h = jnp.maximum(dinv_ref[...] * (agg + xs_ref[...]) + b_ref[...], 0.0)
    out_ref[...] = (
        jnp.dot(h, w_ref[...], preferred_element_type=jnp.float32) * post_ref[...]
    )


def _tc_mid(aggp, xs, dinv, b_row, w, post):
    return pl.pallas_call(
        _tc_mid_body,
        out_shape=jax.ShapeDtypeStruct((N_PAD, H), jnp.float32),
    )(aggp, xs, dinv, b_row, w, post)


BLK = 1024
NB = N_PAD // BLK


def _tc_pool_body(h_ref, bids_ref, wfct_ref, bfc_ref, out_ref, sums, cnt):
    i = pl.program_id(0)

    @pl.when(i == 0)
    def _():
        sums[...] = jnp.zeros_like(sums)
        cnt[...] = jnp.zeros_like(cnt)

    ids = bids_ref[...]
    gi = lax.broadcasted_iota(jnp.int32, (N_GRAPHS, BLK), 0)
    oh = (ids == gi).astype(jnp.float32)
    sums[...] += jnp.dot(oh, h_ref[...], preferred_element_type=jnp.float32)
    cnt[...] += jnp.sum(oh, axis=1, keepdims=True)

    @pl.when(i == NB - 1)
    def _():
        g = sums[...] / jnp.maximum(cnt[...], 1.0)
        out_ref[...] = (
            jnp.dot(g, wfct_ref[...], preferred_element_type=jnp.float32)
            + bfc_ref[...]
        )


def _tc_pool(h, bids, wfct, bfc_row):
    return pl.pallas_call(
        _tc_pool_body,
        grid=(NB,),
        in_specs=[
            pl.BlockSpec((BLK, H), lambda i: (i, 0)),
            pl.BlockSpec((1, BLK), lambda i: (0, i)),
            pl.BlockSpec((H, H), lambda i: (0, 0)),
            pl.BlockSpec((1, H), lambda i: (0, 0)),
        ],
        out_specs=pl.BlockSpec((N_GRAPHS, H), lambda i: (0, 0)),
        out_shape=jax.ShapeDtypeStruct((N_GRAPHS, H), jnp.float32),
        scratch_shapes=[
            pltpu.VMEM((N_GRAPHS, H), jnp.float32),
            pltpu.VMEM((N_GRAPHS, 1), jnp.float32),
        ],
    )(h, bids, wfct, bfc_row)


def kernel(V, E, batch_ids, W1, b1, W2, b2, Wfc, bfc):
    vp = jnp.zeros((N_PAD, IN_DIM), jnp.float32).at[:N_REAL].set(V)
    e = jnp.full((2, E_PAD), PAD_IDX, jnp.int32).at[:, :N_EDGES].set(E)
    esrc = e[0].reshape(NCH_TOT + GUARD, CHUNK)
    edst = e[1].reshape(NCH_TOT + GUARD, CHUNK)
    bids = jnp.full((1, N_PAD), N_GRAPHS, jnp.int32).at[0, :N_REAL].set(batch_ids)

    degp = _sc_deg(edst).reshape(NC, N_PAD, 1)
    xs1, dinv = _tc_a(vp, W1, degp)

    w_stack = jnp.stack([W2, jnp.eye(H, dtype=jnp.float32)])
    b_stack = jnp.stack([b1.reshape(1, H), b2.reshape(1, H)])
    post_stack = jnp.stack([dinv, jnp.ones((N_PAD, 1), jnp.float32)])

    def _layer(xs, wbp):
        w_l, b_l, post_l = wbp
        aggp = _sc_agg(xs, esrc, edst)
        xs_next = _tc_mid(aggp, xs, dinv, b_l, w_l, post_l)
        return xs_next, None

    h2, _ = lax.scan(_layer, xs1, (w_stack, b_stack, post_stack))
    return _tc_pool(h2, bids, Wfc.T, bfc.reshape(1, H))


# speedup vs baseline: 1.0066x; 1.0066x over previous
import functools

import jax
import jax.numpy as jnp
from jax import lax
from jax.experimental import pallas as pl
from jax.experimental.pallas import tpu as pltpu
from jax.experimental.pallas import tpu_sc as plsc

N_REAL = 10000
N_PAD = 10240
N_EDGES = 320000
N_GRAPHS = 512
IN_DIM = 128
H = 64

NC, NS = 2, 16
NW = NC * NS
CHUNK = 128
NCHUNK = 80
NCH_TOT = 2560
SPLIT0 = 120
SPLIT1 = 40
GUARD = SPLIT0 - SPLIT1
E_PAD = (NCH_TOT + GUARD) * CHUNK
ROWS_PER_TILE = N_PAD // NS
PAD_IDX = 10016
NBUF = 5

_MESH = plsc.VectorSubcoreMesh(
    core_axis_name="c", subcore_axis_name="s", num_cores=NC, num_subcores=NS
)
_SC_PARAMS = pltpu.CompilerParams(use_tc_tiling_on_sc=False)


@functools.partial(
    pl.kernel,
    out_type=jax.ShapeDtypeStruct((NC, N_PAD), jnp.float32),
    mesh=_MESH,
    scratch_types=[
        pltpu.VMEM((NCHUNK, CHUNK), jnp.int32),
        pltpu.VMEM((CHUNK,), jnp.float32),
        pltpu.VMEM_SHARED((N_PAD,), jnp.float32),
    ],
    compiler_params=_SC_PARAMS,
)
def _sc_deg(edst_hbm, out_hbm, didx, buf, degsp):
    c = lax.axis_index("c")
    s = lax.axis_index("s")
    wid = c * NS + s
    pltpu.sync_copy(edst_hbm.at[pl.ds(wid * NCHUNK, NCHUNK)], didx)

    def _zero(i, carry):
        buf[pl.ds(i * 16, 16)] = jnp.zeros((16,), jnp.float32)
        return carry

    lax.fori_loop(0, CHUNK // 16, _zero, 0)
    r0 = s * ROWS_PER_TILE
    for j in range(ROWS_PER_TILE // CHUNK):
        pltpu.sync_copy(buf, degsp.at[pl.ds(r0 + j * CHUNK, CHUNK)])
    plsc.subcore_barrier()

    def _ones(i, carry):
        buf[pl.ds(i * 16, 16)] = jnp.ones((16,), jnp.float32)
        return carry

    lax.fori_loop(0, CHUNK // 16, _ones, 0)

    def _scatter(k, carry):
        pltpu.sync_copy(buf, degsp.at[didx.at[k]], add=True)
        return carry

    lax.fori_loop(0, NCHUNK, _scatter, 0)
    plsc.subcore_barrier()
    for j in range(ROWS_PER_TILE // CHUNK):
        pltpu.sync_copy(degsp.at[pl.ds(r0 + j * CHUNK, CHUNK)], buf)
        pltpu.sync_copy(buf, out_hbm.at[c, pl.ds(r0 + j * CHUNK, CHUNK)])


@functools.partial(
    pl.kernel,
    out_type=jax.ShapeDtypeStruct((NC, N_PAD, H), jnp.float32),
    mesh=_MESH,
    scratch_types=(
        [
            pltpu.VMEM((SPLIT0, CHUNK), jnp.int32),
            pltpu.VMEM((SPLIT0, CHUNK), jnp.int32),
        ]
        + [pltpu.VMEM((CHUNK, H), jnp.float32) for _ in range(NBUF)]
        + [
            pltpu.VMEM_SHARED((N_PAD, H), jnp.float32),
        ]
        + [pltpu.SemaphoreType.DMA for _ in range(NBUF)]
    ),
    compiler_params=_SC_PARAMS,
)
def _sc_agg(xs_hbm, esrc_hbm, edst_hbm, out_hbm, sidx, didx, *rest):
    rows = list(rest[:NBUF])
    aggsp = rest[NBUF]
    gsem = list(rest[NBUF + 1:])
    c = lax.axis_index("c")
    s = lax.axis_index("s")
    start = s * (SPLIT0 + SPLIT1) + jnp.where(c == 0, 0, SPLIT0)
    n_my = jnp.where(c == 0, SPLIT0, SPLIT1)
    pltpu.sync_copy(esrc_hbm.at[pl.ds(start, SPLIT0)], sidx)
    pltpu.sync_copy(edst_hbm.at[pl.ds(start, SPLIT0)], didx)

    def _zero(i, carry):
        for j in range(H // 16):
            rows[0][i, pl.ds(j * 16, 16)] = jnp.zeros((16,), jnp.float32)
        return carry

    lax.fori_loop(0, CHUNK, _zero, 0)
    r0 = s * ROWS_PER_TILE
    for j in range(ROWS_PER_TILE // CHUNK):
        pltpu.sync_copy(rows[0], aggsp.at[pl.ds(r0 + j * CHUNK, CHUNK)])
    plsc.subcore_barrier()

    def _group(k4, carry):
        for b in range(NBUF):
            k = k4 * NBUF + b
            kprev = k - NBUF

            @pl.when(kprev >= 0)
            def _():
                pltpu.make_async_copy(
                    xs_hbm.at[sidx.at[kprev]], rows[b], gsem[b]
                ).wait()
                pltpu.sync_copy(rows[b], aggsp.at[didx.at[kprev]], add=True)

            @pl.when(k < n_my)
            def _():
                pltpu.async_copy(xs_hbm.at[sidx.at[k]], rows[b], gsem[b])

        return carry

    lax.fori_loop(0, n_my // NBUF + 1, _group, 0)

    plsc.subcore_barrier()
    for j in range(ROWS_PER_TILE // CHUNK):
        pltpu.sync_copy(aggsp.at[pl.ds(r0 + j * CHUNK, CHUNK)], rows[0])
        pltpu.sync_copy(rows[0], out_hbm.at[c, pl.ds(r0 + j * CHUNK, CHUNK)])


def _tc_a_body(v_ref, w1_ref, degp_ref, xs_ref, dinv_ref):
    d = degp_ref[0, :, 0:1] + degp_ref[1, :, 0:1] + 1.0
    dinv = lax.rsqrt(d)
    dinv_ref[...] = dinv
    xw = jnp.dot(v_ref[...], w1_ref[...], preferred_element_type=jnp.float32)
    xs_ref[...] = xw * dinv


def _tc_a(vp, w1, degp):
    return pl.pallas_call(
        _tc_a_body,
        out_shape=(
            jax.ShapeDtypeStruct((N_PAD, H), jnp.float32),
            jax.ShapeDtypeStruct((N_PAD, 1), jnp.float32),
        ),
    )(vp, w1, degp)


def _tc_b_body(aggp_ref, xs_ref, dinv_ref, b_ref, w_ref, out_ref):
    agg = aggp_ref[0] + aggp_ref[1]
    h = jnp.maximum(dinv_ref[...] * (agg + xs_ref[...]) + b_ref[...], 0.0)
    out_ref[...] = (
        jnp.dot(h, w_ref[...], preferred_element_type=jnp.float32) * dinv_ref[...]
    )


def _tc_b(aggp, xs, dinv, b_row, w):
    return pl.pallas_call(
        _tc_b_body,
        out_shape=jax.ShapeDtypeStruct((N_PAD, H), jnp.float32),
    )(aggp, xs, dinv, b_row, w)


BLK = 1024
NB = N_PAD // BLK


def _tc_pool_body(aggp_ref, xs_ref, dinv_ref, b_ref, bids_ref, wfct_ref,
                  bfc_ref, out_ref, sums, cnt):
    i = pl.program_id(0)

    @pl.when(i == 0)
    def _():
        sums[...] = jnp.zeros_like(sums)
        cnt[...] = jnp.zeros_like(cnt)

    agg = aggp_ref[0] + aggp_ref[1]
    h = jnp.maximum(dinv_ref[...] * (agg + xs_ref[...]) + b_ref[...], 0.0)
    ids = bids_ref[...]
    gi = lax.broadcasted_iota(jnp.int32, (N_GRAPHS, BLK), 0)
    oh = (ids == gi).astype(jnp.float32)
    sums[...] += jnp.dot(oh, h, preferred_element_type=jnp.float32)
    cnt[...] += jnp.sum(oh, axis=1, keepdims=True)

    @pl.when(i == NB - 1)
    def _():
        g = sums[...] / jnp.maximum(cnt[...], 1.0)
        out_ref[...] = (
            jnp.dot(g, wfct_ref[...], preferred_element_type=jnp.float32)
            + bfc_ref[...]
        )


def _tc_pool(aggp, xs, dinv, b_row, bids, wfct, bfc_row):
    return pl.pallas_call(
        _tc_pool_body,
        grid=(NB,),
        in_specs=[
            pl.BlockSpec((NC, BLK, H), lambda i: (0, i, 0)),
            pl.BlockSpec((BLK, H), lambda i: (i, 0)),
            pl.BlockSpec((BLK, 1), lambda i: (i, 0)),
            pl.BlockSpec((1, H), lambda i: (0, 0)),
            pl.BlockSpec((1, BLK), lambda i: (0, i)),
            pl.BlockSpec((H, H), lambda i: (0, 0)),
            pl.BlockSpec((1, H), lambda i: (0, 0)),
        ],
        out_specs=pl.BlockSpec((N_GRAPHS, H), lambda i: (0, 0)),
        out_shape=jax.ShapeDtypeStruct((N_GRAPHS, H), jnp.float32),
        scratch_shapes=[
            pltpu.VMEM((N_GRAPHS, H), jnp.float32),
            pltpu.VMEM((N_GRAPHS, 1), jnp.float32),
        ],
    )(aggp, xs, dinv, b_row, bids, wfct, bfc_row)


def kernel(V, E, batch_ids, W1, b1, W2, b2, Wfc, bfc):
    vp = jnp.zeros((N_PAD, IN_DIM), jnp.float32).at[:N_REAL].set(V)
    e = jnp.full((2, E_PAD), PAD_IDX, jnp.int32).at[:, :N_EDGES].set(E)
    esrc = e[0].reshape(NCH_TOT + GUARD, CHUNK)
    edst = e[1].reshape(NCH_TOT + GUARD, CHUNK)
    bids = jnp.full((1, N_PAD), N_GRAPHS, jnp.int32).at[0, :N_REAL].set(batch_ids)

    degp = _sc_deg(edst).reshape(NC, N_PAD, 1)
    xs1, dinv = _tc_a(vp, W1, degp)
    agg1 = _sc_agg(xs1, esrc, edst)
    xs2 = _tc_b(agg1, xs1, dinv, b1.reshape(1, H), W2)
    agg2 = _sc_agg(xs2, esrc, edst)
    return _tc_pool(agg2, xs2, dinv, b2.reshape(1, H), bids, Wfc.T, bfc.reshape(1, H))


# speedup vs baseline: 1.8343x; 1.8223x over previous
import functools

import jax
import jax.numpy as jnp
from jax import lax
from jax.experimental import pallas as pl
from jax.experimental.pallas import tpu as pltpu
from jax.experimental.pallas import tpu_sc as plsc

N_REAL = 10000
N_PAD = 10240
N_EDGES = 320000
N_GRAPHS = 512
IN_DIM = 128
H = 64

NC, NS = 2, 16
NW = NC * NS
CHUNK = 128
NCHUNK = 80
NCH_TOT = 2560
SPLIT0 = 120
SPLIT1 = 40
GUARD = SPLIT0 - SPLIT1
E_PAD = (NCH_TOT + GUARD) * CHUNK
ROWS_PER_TILE = N_PAD // NS
PAD_IDX = 10016
NBUF = 2

_MESH = plsc.VectorSubcoreMesh(
    core_axis_name="c", subcore_axis_name="s", num_cores=NC, num_subcores=NS
)
_SC_PARAMS = pltpu.CompilerParams(use_tc_tiling_on_sc=False)


@functools.partial(
    pl.kernel,
    out_type=jax.ShapeDtypeStruct((NC, N_PAD), jnp.float32),
    mesh=_MESH,
    scratch_types=[
        pltpu.VMEM((NCHUNK, CHUNK), jnp.int32),
        pltpu.VMEM((CHUNK,), jnp.float32),
        pltpu.VMEM_SHARED((N_PAD,), jnp.float32),
    ],
    compiler_params=_SC_PARAMS,
)
def _sc_deg(edst_hbm, out_hbm, didx, buf, degsp):
    c = lax.axis_index("c")
    s = lax.axis_index("s")
    wid = c * NS + s
    pltpu.sync_copy(edst_hbm.at[pl.ds(wid * NCHUNK, NCHUNK)], didx)

    def _zero(i, carry):
        buf[pl.ds(i * 16, 16)] = jnp.zeros((16,), jnp.float32)
        return carry

    lax.fori_loop(0, CHUNK // 16, _zero, 0)
    r0 = s * ROWS_PER_TILE
    for j in range(ROWS_PER_TILE // CHUNK):
        pltpu.sync_copy(buf, degsp.at[pl.ds(r0 + j * CHUNK, CHUNK)])
    plsc.subcore_barrier()

    def _ones(i, carry):
        buf[pl.ds(i * 16, 16)] = jnp.ones((16,), jnp.float32)
        return carry

    lax.fori_loop(0, CHUNK // 16, _ones, 0)

    def _scatter(k, carry):
        pltpu.sync_copy(buf, degsp.at[didx.at[k]], add=True)
        return carry

    lax.fori_loop(0, NCHUNK, _scatter, 0)
    plsc.subcore_barrier()
    for j in range(ROWS_PER_TILE // CHUNK):
        pltpu.sync_copy(degsp.at[pl.ds(r0 + j * CHUNK, CHUNK)], buf)
        pltpu.sync_copy(buf, out_hbm.at[c, pl.ds(r0 + j * CHUNK, CHUNK)])


NCHA = 80


@functools.partial(
    pl.kernel,
    out_type=jax.ShapeDtypeStruct((NC, N_PAD, H), jnp.float32),
    mesh=_MESH,
    scratch_types=(
        [
            pltpu.VMEM((NCHA, CHUNK), jnp.int32),
            pltpu.VMEM((NCHA, CHUNK), jnp.int32),
        ]
        + [pltpu.VMEM((CHUNK, H), jnp.float32) for _ in range(NBUF)]
        + [
            pltpu.VMEM_SHARED((N_PAD, H), jnp.float32),
            pltpu.VMEM_SHARED((N_PAD, H), jnp.float32),
        ]
        + [pltpu.SemaphoreType.DMA for _ in range(NBUF)]
    ),
    compiler_params=_SC_PARAMS,
)
def _sc_agg(xs_hbm, esrc_hbm, edst_hbm, out_hbm, sidx, didx, *rest):
    rows = list(rest[:NBUF])
    xs_sp = rest[NBUF]
    aggsp = rest[NBUF + 1]
    gsem = list(rest[NBUF + 2:])
    c = lax.axis_index("c")
    s = lax.axis_index("s")
    wid = c * NS + s
    pltpu.sync_copy(esrc_hbm.at[pl.ds(wid * NCHA, NCHA)], sidx)
    pltpu.sync_copy(edst_hbm.at[pl.ds(wid * NCHA, NCHA)], didx)

    def _zero(i, carry):
        for j in range(H // 16):
            rows[0][i, pl.ds(j * 16, 16)] = jnp.zeros((16,), jnp.float32)
        return carry

    lax.fori_loop(0, CHUNK, _zero, 0)
    r0 = s * ROWS_PER_TILE
    for j in range(ROWS_PER_TILE // CHUNK):
        pltpu.sync_copy(rows[0], aggsp.at[pl.ds(r0 + j * CHUNK, CHUNK)])
        pltpu.sync_copy(xs_hbm.at[pl.ds(r0 + j * CHUNK, CHUNK)], rows[1])
        pltpu.sync_copy(rows[1], xs_sp.at[pl.ds(r0 + j * CHUNK, CHUNK)])
    plsc.subcore_barrier()

    def _group(k4, carry):
        for b in range(NBUF):
            k = k4 * NBUF + b
            kprev = k - NBUF

            @pl.when(kprev >= 0)
            def _():
                pltpu.make_async_copy(
                    xs_sp.at[sidx.at[kprev]], rows[b], gsem[b]
                ).wait()
                pltpu.sync_copy(rows[b], aggsp.at[didx.at[kprev]], add=True)

            @pl.when(k < NCHA)
            def _():
                pltpu.async_copy(xs_sp.at[sidx.at[k]], rows[b], gsem[b])

        return carry

    lax.fori_loop(0, NCHA // NBUF + 1, _group, 0)

    plsc.subcore_barrier()
    for j in range(ROWS_PER_TILE // CHUNK):
        pltpu.sync_copy(aggsp.at[pl.ds(r0 + j * CHUNK, CHUNK)], rows[0])
        pltpu.sync_copy(rows[0], out_hbm.at[c, pl.ds(r0 + j * CHUNK, CHUNK)])


def _tc_a_body(v_ref, w1_ref, degp_ref, xs_ref, dinv_ref):
    d = degp_ref[0, :, 0:1] + degp_ref[1, :, 0:1] + 1.0
    dinv = lax.rsqrt(d)
    dinv_ref[...] = dinv
    xw = jnp.dot(v_ref[...], w1_ref[...], preferred_element_type=jnp.float32)
    xs_ref[...] = xw * dinv


def _tc_a(vp, w1, degp):
    return pl.pallas_call(
        _tc_a_body,
        out_shape=(
            jax.ShapeDtypeStruct((N_PAD, H), jnp.float32),
            jax.ShapeDtypeStruct((N_PAD, 1), jnp.float32),
        ),
    )(vp, w1, degp)


def _tc_b_body(aggp_ref, xs_ref, dinv_ref, b_ref, w_ref, out_ref):
    agg = aggp_ref[0] + aggp_ref[1]
    h = jnp.maximum(dinv_ref[...] * (agg + xs_ref[...]) + b_ref[...], 0.0)
    out_ref[...] = (
        jnp.dot(h, w_ref[...], preferred_element_type=jnp.float32) * dinv_ref[...]
    )


def _tc_b(aggp, xs, dinv, b_row, w):
    return pl.pallas_call(
        _tc_b_body,
        out_shape=jax.ShapeDtypeStruct((N_PAD, H), jnp.float32),
    )(aggp, xs, dinv, b_row, w)


BLK = 1024
NB = N_PAD // BLK


def _tc_pool_body(aggp_ref, xs_ref, dinv_ref, b_ref, bids_ref, wfct_ref,
                  bfc_ref, out_ref, sums, cnt):
    i = pl.program_id(0)

    @pl.when(i == 0)
    def _():
        sums[...] = jnp.zeros_like(sums)
        cnt[...] = jnp.zeros_like(cnt)

    agg = aggp_ref[0] + aggp_ref[1]
    h = jnp.maximum(dinv_ref[...] * (agg + xs_ref[...]) + b_ref[...], 0.0)
    ids = bids_ref[...]
    gi = lax.broadcasted_iota(jnp.int32, (N_GRAPHS, BLK), 0)
    oh = (ids == gi).astype(jnp.float32)
    sums[...] += jnp.dot(oh, h, preferred_element_type=jnp.float32)
    cnt[...] += jnp.sum(oh, axis=1, keepdims=True)

    @pl.when(i == NB - 1)
    def _():
        g = sums[...] / jnp.maximum(cnt[...], 1.0)
        out_ref[...] = (
            jnp.dot(g, wfct_ref[...], preferred_element_type=jnp.float32)
            + bfc_ref[...]
        )


def _tc_pool(aggp, xs, dinv, b_row, bids, wfct, bfc_row):
    return pl.pallas_call(
        _tc_pool_body,
        grid=(NB,),
        in_specs=[
            pl.BlockSpec((NC, BLK, H), lambda i: (0, i, 0)),
            pl.BlockSpec((BLK, H), lambda i: (i, 0)),
            pl.BlockSpec((BLK, 1), lambda i: (i, 0)),
            pl.BlockSpec((1, H), lambda i: (0, 0)),
            pl.BlockSpec((1, BLK), lambda i: (0, i)),
            pl.BlockSpec((H, H), lambda i: (0, 0)),
            pl.BlockSpec((1, H), lambda i: (0, 0)),
        ],
        out_specs=pl.BlockSpec((N_GRAPHS, H), lambda i: (0, 0)),
        out_shape=jax.ShapeDtypeStruct((N_GRAPHS, H), jnp.float32),
        scratch_shapes=[
            pltpu.VMEM((N_GRAPHS, H), jnp.float32),
            pltpu.VMEM((N_GRAPHS, 1), jnp.float32),
        ],
    )(aggp, xs, dinv, b_row, bids, wfct, bfc_row)


def kernel(V, E, batch_ids, W1, b1, W2, b2, Wfc, bfc):
    vp = jnp.zeros((N_PAD, IN_DIM), jnp.float32).at[:N_REAL].set(V)
    e = jnp.full((2, E_PAD), PAD_IDX, jnp.int32).at[:, :N_EDGES].set(E)
    esrc = e[0].reshape(NCH_TOT + GUARD, CHUNK)
    edst = e[1].reshape(NCH_TOT + GUARD, CHUNK)
    bids = jnp.full((1, N_PAD), N_GRAPHS, jnp.int32).at[0, :N_REAL].set(batch_ids)

    degp = _sc_deg(edst).reshape(NC, N_PAD, 1)
    xs1, dinv = _tc_a(vp, W1, degp)
    agg1 = _sc_agg(xs1, esrc, edst)
    xs2 = _tc_b(agg1, xs1, dinv, b1.reshape(1, H), W2)
    agg2 = _sc_agg(xs2, esrc, edst)
    return _tc_pool(agg2, xs2, dinv, b2.reshape(1, H), bids, Wfc.T, bfc.reshape(1, H))


# speedup vs baseline: 2.2277x; 1.2145x over previous
import functools

import jax
import jax.numpy as jnp
from jax import lax
from jax.experimental import pallas as pl
from jax.experimental.pallas import tpu as pltpu
from jax.experimental.pallas import tpu_sc as plsc

N_REAL = 10000
N_PAD = 10240
N_EDGES = 320000
N_GRAPHS = 512
IN_DIM = 128
H = 64

NC, NS = 2, 16
NW = NC * NS
CHUNK = 128
NCHUNK = 80
NCH_TOT = NW * NCHUNK
E_PAD = NCH_TOT * CHUNK
ROWS_PER_TILE = N_PAD // NS
PAD_IDX = 10016
NBUF = 2

_MESH = plsc.VectorSubcoreMesh(
    core_axis_name="c", subcore_axis_name="s", num_cores=NC, num_subcores=NS
)
_SC_PARAMS = pltpu.CompilerParams(use_tc_tiling_on_sc=False)


@functools.partial(
    pl.kernel,
    out_type=jax.ShapeDtypeStruct((NC, N_PAD), jnp.float32),
    mesh=_MESH,
    scratch_types=[
        pltpu.VMEM((NCHUNK, CHUNK), jnp.int32),
        pltpu.VMEM((CHUNK,), jnp.float32),
        pltpu.VMEM_SHARED((N_PAD,), jnp.float32),
    ],
    compiler_params=_SC_PARAMS,
)
def _sc_deg(edst_hbm, out_hbm, didx, buf, degsp):
    c = lax.axis_index("c")
    s = lax.axis_index("s")
    wid = c * NS + s
    pltpu.sync_copy(edst_hbm.at[pl.ds(wid * NCHUNK, NCHUNK)], didx)

    def _zero(i, carry):
        buf[pl.ds(i * 16, 16)] = jnp.zeros((16,), jnp.float32)
        return carry

    lax.fori_loop(0, CHUNK // 16, _zero, 0)
    r0 = s * ROWS_PER_TILE
    for j in range(ROWS_PER_TILE // CHUNK):
        pltpu.sync_copy(buf, degsp.at[pl.ds(r0 + j * CHUNK, CHUNK)])
    plsc.subcore_barrier()

    def _ones(i, carry):
        buf[pl.ds(i * 16, 16)] = jnp.ones((16,), jnp.float32)
        return carry

    lax.fori_loop(0, CHUNK // 16, _ones, 0)

    def _scatter(k, carry):
        pltpu.sync_copy(buf, degsp.at[didx.at[k]], add=True)
        return carry

    lax.fori_loop(0, NCHUNK, _scatter, 0)
    plsc.subcore_barrier()
    for j in range(ROWS_PER_TILE // CHUNK):
        pltpu.sync_copy(degsp.at[pl.ds(r0 + j * CHUNK, CHUNK)], buf)
        pltpu.sync_copy(buf, out_hbm.at[c, pl.ds(r0 + j * CHUNK, CHUNK)])


NCHA = 80


@functools.partial(
    pl.kernel,
    out_type=jax.ShapeDtypeStruct((NC, N_PAD, H), jnp.float32),
    mesh=_MESH,
    scratch_types=(
        [
            pltpu.VMEM((NCHA, CHUNK), jnp.int32),
            pltpu.VMEM((NCHA, CHUNK), jnp.int32),
        ]
        + [pltpu.VMEM((CHUNK, H), jnp.float32) for _ in range(NBUF)]
        + [
            pltpu.VMEM_SHARED((N_PAD, H), jnp.float32),
            pltpu.VMEM_SHARED((N_PAD, H), jnp.float32),
        ]
        + [pltpu.SemaphoreType.DMA for _ in range(NBUF)]
    ),
    compiler_params=_SC_PARAMS,
)
def _sc_agg(xs_hbm, esrc_hbm, edst_hbm, out_hbm, sidx, didx, *rest):
    rows = list(rest[:NBUF])
    xs_sp = rest[NBUF]
    aggsp = rest[NBUF + 1]
    gsem = list(rest[NBUF + 2:])
    c = lax.axis_index("c")
    s = lax.axis_index("s")
    wid = c * NS + s
    pltpu.sync_copy(esrc_hbm.at[pl.ds(wid * NCHA, NCHA)], sidx)
    pltpu.sync_copy(edst_hbm.at[pl.ds(wid * NCHA, NCHA)], didx)

    def _zero(i, carry):
        for j in range(H // 16):
            rows[0][i, pl.ds(j * 16, 16)] = jnp.zeros((16,), jnp.float32)
        return carry

    lax.fori_loop(0, CHUNK, _zero, 0)
    r0 = s * ROWS_PER_TILE
    for j in range(ROWS_PER_TILE // CHUNK):
        pltpu.sync_copy(rows[0], aggsp.at[pl.ds(r0 + j * CHUNK, CHUNK)])
        pltpu.sync_copy(xs_hbm.at[pl.ds(r0 + j * CHUNK, CHUNK)], rows[1])
        pltpu.sync_copy(rows[1], xs_sp.at[pl.ds(r0 + j * CHUNK, CHUNK)])
    plsc.subcore_barrier()

    def _group(k4, carry):
        for b in range(NBUF):
            k = k4 * NBUF + b
            kprev = k - NBUF

            @pl.when(kprev >= 0)
            def _():
                pltpu.make_async_copy(
                    xs_sp.at[sidx.at[kprev]], rows[b], gsem[b]
                ).wait()
                pltpu.sync_copy(rows[b], aggsp.at[didx.at[kprev]], add=True)

            @pl.when(k < NCHA)
            def _():
                pltpu.async_copy(xs_sp.at[sidx.at[k]], rows[b], gsem[b])

        return carry

    lax.fori_loop(0, NCHA // NBUF + 1, _group, 0)

    plsc.subcore_barrier()
    for j in range(ROWS_PER_TILE // CHUNK):
        pltpu.sync_copy(aggsp.at[pl.ds(r0 + j * CHUNK, CHUNK)], rows[0])
        pltpu.sync_copy(rows[0], out_hbm.at[c, pl.ds(r0 + j * CHUNK, CHUNK)])


NP2 = N_PAD // 2


def _tc_mm_body(v_ref, w1_ref, xw_ref):
    xw_ref[...] = jnp.dot(
        v_ref[...], w1_ref[...], preferred_element_type=jnp.float32
    )


def _tc_mm(vp2, w1bd):
    return pl.pallas_call(
        _tc_mm_body,
        out_shape=jax.ShapeDtypeStruct((NP2, 2 * H), jnp.float32),
    )(vp2, w1bd)


def _tc_scale_body(xw_ref, deg2_ref, sel_ref, xs_ref, dinv_ref):
    dinv2 = lax.rsqrt(deg2_ref[...] + 1.0)
    dinv = jnp.dot(dinv2, sel_ref[...], preferred_element_type=jnp.float32)
    dinv_ref[...] = dinv
    xs_ref[...] = xw_ref[...] * dinv


def _tc_scale(xw, deg2, sel):
    return pl.pallas_call(
        _tc_scale_body,
        out_shape=(
            jax.ShapeDtypeStruct((NP2, 2 * H), jnp.float32),
            jax.ShapeDtypeStruct((NP2, 2 * H), jnp.float32),
        ),
    )(xw, deg2, sel)


def _tc_b_body(aggp_ref, xs_ref, dinv_ref, b_ref, w_ref, out_ref):
    agg = aggp_ref[0] + aggp_ref[1]
    h = jnp.maximum(dinv_ref[...] * (agg + xs_ref[...]) + b_ref[...], 0.0)
    out_ref[...] = (
        jnp.dot(h, w_ref[...], preferred_element_type=jnp.float32) * dinv_ref[...]
    )


def _tc_b(aggp, xs, dinv, b_row, w):
    return pl.pallas_call(
        _tc_b_body,
        out_shape=jax.ShapeDtypeStruct((NP2, 2 * H), jnp.float32),
    )(aggp, xs, dinv, b_row, w)


BLK = 1024
NB = NP2 // BLK


def _tc_pool_body(aggp_ref, xs_ref, dinv_ref, b_ref, bidse_ref, bidso_ref,
                  wfct_ref, bfc_ref, out_ref, sums, cnt):
    i = pl.program_id(0)

    @pl.when(i == 0)
    def _():
        sums[...] = jnp.zeros_like(sums)
        cnt[...] = jnp.zeros_like(cnt)

    agg = aggp_ref[0] + aggp_ref[1]
    h = jnp.maximum(dinv_ref[...] * (agg + xs_ref[...]) + b_ref[...], 0.0)
    gi = lax.broadcasted_iota(jnp.int32, (N_GRAPHS, BLK), 0)
    ohe = (bidse_ref[...] == gi).astype(jnp.float32)
    oho = (bidso_ref[...] == gi).astype(jnp.float32)
    sums[...] += jnp.dot(ohe, h[:, :H], preferred_element_type=jnp.float32)
    sums[...] += jnp.dot(oho, h[:, H:], preferred_element_type=jnp.float32)
    cnt[...] += jnp.sum(ohe, axis=1, keepdims=True)
    cnt[...] += jnp.sum(oho, axis=1, keepdims=True)

    @pl.when(i == NB - 1)
    def _():
        g = sums[...] / jnp.maximum(cnt[...], 1.0)
        out_ref[...] = (
            jnp.dot(g, wfct_ref[...], preferred_element_type=jnp.float32)
            + bfc_ref[...]
        )


def _tc_pool(aggp, xs, dinv, b_row, bidse, bidso, wfct, bfc_row):
    return pl.pallas_call(
        _tc_pool_body,
        grid=(NB,),
        in_specs=[
            pl.BlockSpec((NC, BLK, 2 * H), lambda i: (0, i, 0)),
            pl.BlockSpec((BLK, 2 * H), lambda i: (i, 0)),
            pl.BlockSpec((BLK, 2 * H), lambda i: (i, 0)),
            pl.BlockSpec((1, 2 * H), lambda i: (0, 0)),
            pl.BlockSpec((1, BLK), lambda i: (0, i)),
            pl.BlockSpec((1, BLK), lambda i: (0, i)),
            pl.BlockSpec((H, H), lambda i: (0, 0)),
            pl.BlockSpec((1, H), lambda i: (0, 0)),
        ],
        out_specs=pl.BlockSpec((N_GRAPHS, H), lambda i: (0, 0)),
        out_shape=jax.ShapeDtypeStruct((N_GRAPHS, H), jnp.float32),
        scratch_shapes=[
            pltpu.VMEM((N_GRAPHS, H), jnp.float32),
            pltpu.VMEM((N_GRAPHS, 1), jnp.float32),
        ],
    )(aggp, xs, dinv, b_row, bidse, bidso, wfct, bfc_row)


def kernel(V, E, batch_ids, W1, b1, W2, b2, Wfc, bfc):
    vp2 = (
        jnp.zeros((N_PAD, IN_DIM), jnp.float32).at[:N_REAL].set(V)
        .reshape(NP2, 2 * IN_DIM)
    )
    e = jnp.full((2, E_PAD), PAD_IDX, jnp.int32).at[:, :N_EDGES].set(E)
    esrc = e[0].reshape(NCH_TOT, CHUNK)
    edst = e[1].reshape(NCH_TOT, CHUNK)
    bids = jnp.full((N_PAD,), N_GRAPHS, jnp.int32).at[:N_REAL].set(batch_ids)
    bidse = bids[0::2].reshape(1, NP2)
    bidso = bids[1::2].reshape(1, NP2)

    w1bd = (
        jnp.zeros((2 * IN_DIM, 2 * H), jnp.float32)
        .at[:IN_DIM, :H].set(W1)
        .at[IN_DIM:, H:].set(W1)
    )
    w2bd = (
        jnp.zeros((2 * H, 2 * H), jnp.float32)
        .at[:H, :H].set(W2)
        .at[H:, H:].set(W2)
    )
    sel = jnp.zeros((2, 2 * H), jnp.float32).at[0, :H].set(1.0).at[1, H:].set(1.0)
    b1p = jnp.concatenate([b1, b1]).reshape(1, 2 * H)
    b2p = jnp.concatenate([b2, b2]).reshape(1, 2 * H)

    xw1 = _tc_mm(vp2, w1bd)
    dl = _sc_deg(edst)
    deg2 = (dl[0] + dl[1]).reshape(NP2, 2)
    xs1, dinv = _tc_scale(xw1, deg2, sel)
    agg1 = _sc_agg(xs1.reshape(N_PAD, H), esrc, edst).reshape(NC, NP2, 2 * H)
    xs2 = _tc_b(agg1, xs1, dinv, b1p, w2bd)
    agg2 = _sc_agg(xs2.reshape(N_PAD, H), esrc, edst).reshape(NC, NP2, 2 * H)
    return _tc_pool(agg2, xs2, dinv, b2p, bidse, bidso, Wfc.T, bfc.reshape(1, H))
